# Initial kernel scaffold; baseline (speedup 1.0000x reference)
import jax
import jax.numpy as jnp
from jax.experimental import pallas as pl


def kernel(inputs_embeds, token_type_ids, pos_table, type_table, ln_gamma, ln_beta):
    raise NotImplementedError("write your pallas kernel here")


# speedup vs baseline: 3.7089x; 3.7089x over previous
import jax
import jax.numpy as jnp
from jax.experimental import pallas as pl
from jax.experimental.pallas import tpu as pltpu

_MODALITY = (197, 50, 50, 50, 200, 105, 277, 61, 34)
_B, _S, _H = 16, 1024, 1024
_NTYPE = 9
_NTYPE_PAD = 16
_EPS = 1e-12
_POS_ROWS = 512


def _fused_kernel(tt_ref, x_ref, type_ref, pos_ref, gamma_ref, beta_ref,
                  o_ref, pos_emb_ref):
    b = pl.program_id(0)

    @pl.when(b == 0)
    def _fill_pos():
        off = 0
        for L in _MODALITY:
            pos_emb_ref[off:off + L, :] = pos_ref[0:L, :]
            off += L

    x = x_ref[0]
    ids = tt_ref[0]
    iota = jax.lax.broadcasted_iota(jnp.int32, (_NTYPE_PAD, _S), 0)
    onehot = (iota == ids).astype(jnp.float32)
    type_emb = jax.lax.dot_general(
        onehot, type_ref[...],
        dimension_numbers=(((0,), (0,)), ((), ())),
        preferred_element_type=jnp.float32)

    s = x + type_emb + pos_emb_ref[...]
    mean = jnp.mean(s, axis=1, keepdims=True)
    c = s - mean
    var = jnp.mean(c * c, axis=1, keepdims=True)
    inv = jax.lax.rsqrt(var + _EPS)
    o_ref[0] = c * inv * gamma_ref[...] + beta_ref[...]


def kernel(inputs_embeds, token_type_ids, pos_table, type_table, ln_gamma, ln_beta):
    tt = token_type_ids.astype(jnp.int32).reshape(_B, 1, _S)
    type_pad = jnp.zeros((_NTYPE_PAD, _H), jnp.float32).at[:_NTYPE].set(
        type_table.astype(jnp.float32))
    gamma = ln_gamma.reshape(1, _H)
    beta = ln_beta.reshape(1, _H)

    out = pl.pallas_call(
        _fused_kernel,
        grid=(_B,),
        in_specs=[
            pl.BlockSpec((1, 1, _S), lambda b: (b, 0, 0)),
            pl.BlockSpec((1, _S, _H), lambda b: (b, 0, 0)),
            pl.BlockSpec((_NTYPE_PAD, _H), lambda b: (0, 0)),
            pl.BlockSpec((_POS_ROWS, _H), lambda b: (0, 0)),
            pl.BlockSpec((1, _H), lambda b: (0, 0)),
            pl.BlockSpec((1, _H), lambda b: (0, 0)),
        ],
        out_specs=pl.BlockSpec((1, _S, _H), lambda b: (b, 0, 0)),
        out_shape=jax.ShapeDtypeStruct((_B, _S, _H), jnp.float32),
        scratch_shapes=[pltpu.VMEM((_S, _H), jnp.float32)],
    )(tt, inputs_embeds, type_pad, pos_table[:_POS_ROWS], gamma, beta)
    return out
